# Initial kernel scaffold; baseline (speedup 1.0000x reference)
import functools
import jax
import jax.numpy as jnp
from jax import lax
from jax.experimental import pallas as pl
from jax.experimental.pallas import tpu as pltpu
from jax.experimental.pallas import tpu_sc as plsc

D = 64
NC, NS = 2, 16
NW = NC * NS
CHUNK = 128
GRP = 4
NBUF = 2 * GRP


@functools.lru_cache(maxsize=None)
def _build(B):
    assert B % (NW * CHUNK) == 0
    b_per_w = B // NW
    n_chunks = b_per_w // CHUNK
    n_outer = n_chunks // GRP
    assert n_chunks % GRP == 0 and n_outer >= 2

    mesh = plsc.VectorSubcoreMesh(core_axis_name="c", subcore_axis_name="s")

    @functools.partial(
        pl.kernel,
        mesh=mesh,
        out_type=jax.ShapeDtypeStruct((B, D), jnp.float32),
        scratch_types=[
            pltpu.VMEM((n_chunks, CHUNK), jnp.int32),
            pltpu.VMEM((NBUF, CHUNK, D), jnp.float32),
            pltpu.SemaphoreType.DMA,
            pltpu.SemaphoreType.DMA,
        ],
    )
    def k(idx_hbm, table_hbm, out_hbm, idx_v, rows_v, gsem, wsem):
        wid = lax.axis_index("s") * NC + lax.axis_index("c")
        base = wid * b_per_w
        pltpu.sync_copy(idx_hbm.at[wid], idx_v)

        def outer(o, carry):
            grp = lax.rem(o, 2)
            @pl.when(o >= 2)
            def _():
                for j in range(GRP):
                    pltpu.make_async_copy(
                        rows_v.at[j], out_hbm.at[pl.ds(0, CHUNK)], wsem
                    ).wait()

            copies = []
            for j in range(GRP):
                g = o * GRP + j
                buf = grp * GRP + j
                copies.append(
                    pltpu.async_copy(
                        table_hbm.at[idx_v.at[g]], rows_v.at[buf], gsem
                    )
                )
            for c in copies:
                c.wait()
            for j in range(GRP):
                g = o * GRP + j
                buf = grp * GRP + j
                pltpu.async_copy(
                    rows_v.at[buf], out_hbm.at[pl.ds(base + g * CHUNK, CHUNK)],
                    wsem,
                )
            return carry

        lax.fori_loop(0, n_outer, outer, 0, unroll=False)
        for j in range(NBUF):
            pltpu.make_async_copy(
                rows_v.at[j % GRP], out_hbm.at[pl.ds(0, CHUNK)], wsem
            ).wait()

    return k


def kernel(x, table):
    R, C = x.shape
    B = R * C
    idx = x.reshape(B).astype(jnp.int32)
    b_per_w = B // NW
    idx3 = idx.reshape(NW, b_per_w // CHUNK, CHUNK)
    out = _build(B)(idx3, table)
    return out.reshape(R, C, D)


# speedup vs baseline: 1.8706x; 1.8706x over previous
import functools
import jax
import jax.numpy as jnp
from jax import lax
from jax.experimental import pallas as pl
from jax.experimental.pallas import tpu as pltpu
from jax.experimental.pallas import tpu_sc as plsc

D = 64
NC, NS = 2, 16
NW = NC * NS
CHUNK = 128
GRP = 4
NBUF = 2 * GRP


@functools.lru_cache(maxsize=None)
def _build(B):
    assert B % (NW * CHUNK) == 0
    b_per_w = B // NW
    n_chunks = b_per_w // CHUNK
    n_outer = n_chunks // GRP
    assert n_chunks % GRP == 0 and n_outer >= 2

    mesh = plsc.VectorSubcoreMesh(core_axis_name="c", subcore_axis_name="s")

    @functools.partial(
        pl.kernel,
        mesh=mesh,
        compiler_params=pltpu.CompilerParams(use_tc_tiling_on_sc=False),
        out_type=jax.ShapeDtypeStruct((B, D), jnp.float32),
        scratch_types=[
            pltpu.VMEM((n_chunks, CHUNK), jnp.int32),
            pltpu.VMEM((NBUF, CHUNK, D), jnp.float32),
            pltpu.SemaphoreType.DMA,
            pltpu.SemaphoreType.DMA,
        ],
    )
    def k(idx_hbm, table_hbm, out_hbm, idx_v, rows_v, gsem, wsem):
        wid = lax.axis_index("s") * NC + lax.axis_index("c")
        base = wid * b_per_w
        pltpu.sync_copy(idx_hbm.at[wid], idx_v)

        def outer(o, carry):
            grp = lax.rem(o, 2)
            @pl.when(o >= 2)
            def _():
                for j in range(GRP):
                    pltpu.make_async_copy(
                        rows_v.at[j], out_hbm.at[pl.ds(0, CHUNK)], wsem
                    ).wait()

            copies = []
            for j in range(GRP):
                g = o * GRP + j
                buf = grp * GRP + j
                copies.append(
                    pltpu.async_copy(
                        table_hbm.at[idx_v.at[g]], rows_v.at[buf], gsem
                    )
                )
            for c in copies:
                c.wait()
            for j in range(GRP):
                g = o * GRP + j
                buf = grp * GRP + j
                pltpu.async_copy(
                    rows_v.at[buf], out_hbm.at[pl.ds(base + g * CHUNK, CHUNK)],
                    wsem,
                )
            return carry

        lax.fori_loop(0, n_outer, outer, 0, unroll=False)
        for j in range(NBUF):
            pltpu.make_async_copy(
                rows_v.at[j % GRP], out_hbm.at[pl.ds(0, CHUNK)], wsem
            ).wait()

    return k


def kernel(x, table):
    R, C = x.shape
    B = R * C
    idx = x.reshape(B).astype(jnp.int32)
    b_per_w = B // NW
    idx3 = idx.reshape(NW, b_per_w // CHUNK, CHUNK)
    out = _build(B)(idx3, table)
    return out.reshape(R, C, D)


# speedup vs baseline: 1.8732x; 1.0014x over previous
import functools
import jax
import jax.numpy as jnp
from jax import lax
from jax.experimental import pallas as pl
from jax.experimental.pallas import tpu as pltpu
from jax.experimental.pallas import tpu_sc as plsc

D = 64
NC, NS = 2, 16
NW = NC * NS
CHUNK = 128
GRP = 5
NBUF = 2 * GRP


@functools.lru_cache(maxsize=None)
def _build(B):
    assert B % (NW * CHUNK) == 0
    b_per_w = B // NW
    n_chunks = b_per_w // CHUNK
    n_outer = n_chunks // GRP
    assert n_chunks % GRP == 0 and n_outer >= 2

    mesh = plsc.VectorSubcoreMesh(core_axis_name="c", subcore_axis_name="s")

    @functools.partial(
        pl.kernel,
        mesh=mesh,
        compiler_params=pltpu.CompilerParams(use_tc_tiling_on_sc=False),
        out_type=jax.ShapeDtypeStruct((B, D), jnp.float32),
        scratch_types=[
            pltpu.VMEM((n_chunks, CHUNK), jnp.int32),
            pltpu.VMEM((NBUF, CHUNK, D), jnp.float32),
            pltpu.SemaphoreType.DMA,
            pltpu.SemaphoreType.DMA,
        ],
    )
    def k(idx_hbm, table_hbm, out_hbm, idx_v, rows_v, gsem, wsem):
        wid = lax.axis_index("s") * NC + lax.axis_index("c")
        base = wid * b_per_w
        pltpu.sync_copy(idx_hbm.at[wid], idx_v)

        def outer(o, carry):
            grp = lax.rem(o, 2)
            @pl.when(o >= 2)
            def _():
                for j in range(GRP):
                    pltpu.make_async_copy(
                        rows_v.at[j], out_hbm.at[pl.ds(0, CHUNK)], wsem
                    ).wait()

            copies = []
            for j in range(GRP):
                g = o * GRP + j
                buf = grp * GRP + j
                copies.append(
                    pltpu.async_copy(
                        table_hbm.at[idx_v.at[g]], rows_v.at[buf], gsem
                    )
                )
            for c in copies:
                c.wait()
            for j in range(GRP):
                g = o * GRP + j
                buf = grp * GRP + j
                pltpu.async_copy(
                    rows_v.at[buf], out_hbm.at[pl.ds(base + g * CHUNK, CHUNK)],
                    wsem,
                )
            return carry

        lax.fori_loop(0, n_outer, outer, 0, unroll=False)
        for j in range(NBUF):
            pltpu.make_async_copy(
                rows_v.at[j % GRP], out_hbm.at[pl.ds(0, CHUNK)], wsem
            ).wait()

    return k


def kernel(x, table):
    R, C = x.shape
    B = R * C
    idx = x.reshape(B).astype(jnp.int32)
    b_per_w = B // NW
    idx3 = idx.reshape(NW, b_per_w // CHUNK, CHUNK)
    out = _build(B)(idx3, table)
    return out.reshape(R, C, D)


# speedup vs baseline: 1.9552x; 1.0438x over previous
import functools
import jax
import jax.numpy as jnp
from jax import lax
from jax.experimental import pallas as pl
from jax.experimental.pallas import tpu as pltpu
from jax.experimental.pallas import tpu_sc as plsc

D = 64
NC, NS = 2, 16
NW = NC * NS
CHUNK = 128
GRP = 5
NBUF = 2 * GRP


@functools.lru_cache(maxsize=None)
def _build(B):
    assert B % (NW * CHUNK) == 0
    b_per_w = B // NW
    n_chunks = b_per_w // CHUNK
    n_outer = n_chunks // GRP
    assert n_chunks % GRP == 0 and n_outer >= 2

    mesh = plsc.VectorSubcoreMesh(core_axis_name="c", subcore_axis_name="s")

    @functools.partial(
        pl.kernel,
        mesh=mesh,
        compiler_params=pltpu.CompilerParams(use_tc_tiling_on_sc=False),
        out_type=jax.ShapeDtypeStruct((B, D), jnp.float32),
        scratch_types=[
            pltpu.VMEM((n_chunks, CHUNK), jnp.int32),
            pltpu.VMEM((NBUF, CHUNK, D), jnp.float32),
            pltpu.SemaphoreType.DMA,
            pltpu.SemaphoreType.DMA,
        ],
    )
    def k(idx_hbm, table_hbm, out_hbm, idx_v, rows_v, gsem, wsem):
        wid = lax.axis_index("s") * NC + lax.axis_index("c")
        base = wid * b_per_w
        pltpu.sync_copy(idx_hbm.at[wid], idx_v)

        def outer(o, carry):
            grp = lax.rem(o, 2)
            @pl.when(o >= 2)
            def _():
                for j in range(GRP):
                    pltpu.make_async_copy(
                        rows_v.at[j], out_hbm.at[pl.ds(0, CHUNK)], wsem
                    ).wait()

            copies = []
            for j in range(GRP):
                g = o * GRP + j
                buf = grp * GRP + j
                copies.append(
                    pltpu.async_copy(
                        table_hbm.at[idx_v.at[g]], rows_v.at[buf], gsem
                    )
                )
            for c in copies:
                c.wait()
            for j in range(GRP):
                g = o * GRP + j
                buf = grp * GRP + j
                pltpu.async_copy(
                    rows_v.at[buf], out_hbm.at[pl.ds(base + g * CHUNK, CHUNK)],
                    wsem,
                )
            return carry

        lax.fori_loop(0, n_outer, outer, 0, unroll=False)
        for j in range(NBUF):
            pltpu.make_async_copy(
                rows_v.at[j % GRP], out_hbm.at[pl.ds(0, CHUNK)], wsem
            ).wait()

    return k


def kernel(x, table):
    R, C = x.shape
    B = R * C
    xt = x.T.astype(jnp.int32)
    b_per_w = B // NW
    idx3 = xt.reshape(NW, b_per_w // CHUNK, CHUNK)
    out = _build(B)(idx3, table)
    return out.reshape(C, R, D).transpose(1, 0, 2)
